# Initial kernel scaffold; baseline (speedup 1.0000x reference)
import jax
import jax.numpy as jnp
from jax.experimental import pallas as pl


def kernel(x, edge_index, edge_attr, W1, b1, W2, b2, Wih, Whh, bih, bhh):
    raise NotImplementedError("write your pallas kernel here")


# speedup vs baseline: 2.9291x; 2.9291x over previous
import functools

import jax
import jax.numpy as jnp
from jax import lax
from jax.experimental import pallas as pl
from jax.experimental.pallas import tpu as pltpu
from jax.experimental.pallas import tpu_sc as plsc

_NC = 2
_NS = 16
_NW = _NC * _NS
_CHUNK = 80


def _linear_body(x_ref, w_ref, b_ref, o_ref):
    o_ref[...] = (
        jnp.dot(x_ref[...], w_ref[...], preferred_element_type=jnp.float32)
        + b_ref[...]
    )


def _node_linear(x, wcat, bcat, block_rows, interpret=False):
    n, h = x.shape
    ho = wcat.shape[1]
    return pl.pallas_call(
        _linear_body,
        grid=(n // block_rows,),
        in_specs=[
            pl.BlockSpec((block_rows, h), lambda i: (i, 0)),
            pl.BlockSpec((h, ho), lambda i: (0, 0)),
            pl.BlockSpec((1, ho), lambda i: (0, 0)),
        ],
        out_specs=pl.BlockSpec((block_rows, ho), lambda i: (i, 0)),
        out_shape=jax.ShapeDtypeStruct((n, ho), jnp.float32),
        interpret=interpret,
    )(x, wcat, bcat)


def _edge_mlp_body(gs_ref, gd_ref, ea_ref, wc_ref, b1_ref, w2_ref, b2_ref, o_ref):
    pre = (
        gs_ref[...]
        + gd_ref[...]
        + jnp.dot(ea_ref[...], wc_ref[...], preferred_element_type=jnp.float32)
        + b1_ref[...]
    )
    hdn = jnp.maximum(pre, 0.0)
    o_ref[...] = (
        jnp.dot(hdn, w2_ref[...], preferred_element_type=jnp.float32) + b2_ref[...]
    )


def _edge_mlp(gs, gd, ea, wc, b1, w2, b2, block_rows, interpret=False):
    e, h = ea.shape
    row_spec = pl.BlockSpec((block_rows, h), lambda i: (i, 0))
    full = lambda shape: pl.BlockSpec(shape, lambda i: (0, 0))
    return pl.pallas_call(
        _edge_mlp_body,
        grid=(e // block_rows,),
        in_specs=[
            row_spec, row_spec, row_spec,
            full((h, h)), full((1, h)), full((h, h)), full((1, h)),
        ],
        out_specs=row_spec,
        out_shape=jax.ShapeDtypeStruct((e, h), jnp.float32),
        interpret=interpret,
    )(gs, gd, ea, wc, b1, w2, b2)


def _gru_body(a0_ref, a1_ref, gh_ref, x_ref, wih_ref, bih_ref, o_ref):
    h = x_ref.shape[1]
    agg = a0_ref[...] + a1_ref[...]
    gi = (
        jnp.dot(agg, wih_ref[...], preferred_element_type=jnp.float32)
        + bih_ref[...]
    )
    gh = gh_ref[...]
    r = jax.nn.sigmoid(gi[:, :h] + gh[:, :h])
    z = jax.nn.sigmoid(gi[:, h:2 * h] + gh[:, h:2 * h])
    nn = jnp.tanh(gi[:, 2 * h:] + r * gh[:, 2 * h:])
    o_ref[...] = (1.0 - z) * nn + z * x_ref[...]


def _gru(a0, a1, gh, x, wih, bih, block_rows, interpret=False):
    n, h = x.shape
    row_spec = pl.BlockSpec((block_rows, h), lambda i: (i, 0))
    row3_spec = pl.BlockSpec((block_rows, 3 * h), lambda i: (i, 0))
    full = lambda shape: pl.BlockSpec(shape, lambda i: (0, 0))
    return pl.pallas_call(
        _gru_body,
        grid=(n // block_rows,),
        in_specs=[
            row_spec, row_spec, row3_spec, row_spec,
            full((h, 3 * h)), full((1, 3 * h)),
        ],
        out_specs=row_spec,
        out_shape=jax.ShapeDtypeStruct((n, h), jnp.float32),
        interpret=interpret,
    )(a0, a1, gh, x, wih, bih)


def _make_gather(n, e, h):
    epw = e // _NW
    nch = epw // _CHUNK
    mesh = plsc.VectorSubcoreMesh(core_axis_name="c", subcore_axis_name="s")

    @functools.partial(
        pl.kernel,
        out_type=(
            jax.ShapeDtypeStruct((e, h), jnp.float32),
            jax.ShapeDtypeStruct((e, h), jnp.float32),
        ),
        mesh=mesh,
        scratch_types=[
            pltpu.VMEM((_CHUNK,), jnp.int32),
            pltpu.VMEM((_CHUNK,), jnp.int32),
            pltpu.VMEM((_CHUNK, h), jnp.float32),
            pltpu.VMEM((_CHUNK, h), jnp.float32),
            pltpu.SemaphoreType.DMA,
            pltpu.SemaphoreType.DMA,
        ],
    )
    def gather(xa_hbm, xb_hbm, src_hbm, dst_hbm, gs_hbm, gd_hbm,
               sidx, didx, srows, drows, sem_a, sem_b):
        wid = lax.axis_index("s") * _NC + lax.axis_index("c")
        base0 = wid * epw

        @pl.loop(0, nch)
        def _chunk(j):
            base = base0 + j * _CHUNK
            pltpu.sync_copy(src_hbm.at[pl.ds(base, _CHUNK)], sidx)
            pltpu.sync_copy(dst_hbm.at[pl.ds(base, _CHUNK)], didx)
            ca = pltpu.async_copy(xa_hbm.at[sidx], srows, sem_a)
            cb = pltpu.async_copy(xb_hbm.at[didx], drows, sem_b)
            ca.wait()
            cb.wait()
            pltpu.sync_copy(srows, gs_hbm.at[pl.ds(base, _CHUNK)])
            pltpu.sync_copy(drows, gd_hbm.at[pl.ds(base, _CHUNK)])

    return gather


def _make_scatter(n, e, h):
    epw = e // _NW
    nch = epw // _CHUNK
    rpt = (n // _NS) & ~7
    tail = n - _NS * rpt
    mesh = plsc.VectorSubcoreMesh(core_axis_name="c", subcore_axis_name="s")

    @functools.partial(
        pl.kernel,
        out_type=jax.ShapeDtypeStruct((_NC * n, h), jnp.float32),
        mesh=mesh,
        scratch_types=[
            pltpu.VMEM((_CHUNK,), jnp.int32),
            pltpu.VMEM((_CHUNK, h), jnp.float32),
            pltpu.VMEM_SHARED((n, h), jnp.float32),
        ],
    )
    def scatter(msg_hbm, dst_hbm, zero_hbm, out_hbm, idx, rows, acc):
        cid = lax.axis_index("c")
        sid = lax.axis_index("s")
        wid = sid * _NC + cid
        base0 = wid * epw
        pltpu.sync_copy(
            zero_hbm.at[pl.ds(sid * rpt, rpt)],
            acc.at[pl.ds(sid * rpt, rpt)],
        )
        if tail:
            @pl.when(sid == 0)
            def _zero_tail():
                pltpu.sync_copy(
                    zero_hbm.at[pl.ds(_NS * rpt, tail)],
                    acc.at[pl.ds(_NS * rpt, tail)],
                )
        plsc.subcore_barrier()

        @pl.loop(0, nch)
        def _chunk(j):
            base = base0 + j * _CHUNK
            pltpu.sync_copy(dst_hbm.at[pl.ds(base, _CHUNK)], idx)
            pltpu.sync_copy(msg_hbm.at[pl.ds(base, _CHUNK)], rows)
            pltpu.sync_copy(rows, acc.at[idx], add=True)

        plsc.subcore_barrier()
        pltpu.sync_copy(
            acc.at[pl.ds(sid * rpt, rpt)],
            out_hbm.at[pl.ds(cid * n + sid * rpt, rpt)],
        )
        if tail:
            @pl.when(sid == 0)
            def _out_tail():
                pltpu.sync_copy(
                    acc.at[pl.ds(_NS * rpt, tail)],
                    out_hbm.at[pl.ds(cid * n + _NS * rpt, tail)],
                )

    return scatter


def kernel(x, edge_index, edge_attr, W1, b1, W2, b2, Wih, Whh, bih, bhh):
    n, h = x.shape
    e = edge_index.shape[1]
    src = edge_index[0].astype(jnp.int32)
    dst = edge_index[1].astype(jnp.int32)

    wa = W1[:, :h].T
    wb = W1[:, h:2 * h].T
    wc = W1[:, 2 * h:].T
    w2t = W2.T
    whht = Whh.T
    wiht = Wih.T
    wcat = jnp.concatenate([wa, wb, whht], axis=1)
    bcat = jnp.concatenate(
        [jnp.zeros((2 * h,), jnp.float32), bhh]
    ).reshape(1, 5 * h)

    node_out = _node_linear(x, wcat, bcat, block_rows=2000)
    xa = node_out[:, :h]
    xb = node_out[:, h:2 * h]
    gh = node_out[:, 2 * h:]

    gs, gd = _make_gather(n, e, h)(xa, xb, src, dst)

    msg = _edge_mlp(
        gs, gd, edge_attr, wc, b1.reshape(1, h), w2t, b2.reshape(1, h),
        block_rows=2000,
    )

    zeros = jnp.zeros((n, h), jnp.float32)
    agg2 = _make_scatter(n, e, h)(msg, dst, zeros)

    return _gru(
        agg2[:n], agg2[n:], gh, x, wiht, bih.reshape(1, 3 * h),
        block_rows=2000,
    )


# speedup vs baseline: 3.3431x; 1.1413x over previous
import functools

import jax
import jax.numpy as jnp
from jax import lax
from jax.experimental import pallas as pl
from jax.experimental.pallas import tpu as pltpu
from jax.experimental.pallas import tpu_sc as plsc

_NC = 2
_NS = 16
_NW = _NC * _NS
_CHUNK = 80


def _linear_body(x_ref, w_ref, b_ref, gh_ref, xa_ref, xb_ref):
    h = x_ref.shape[1]
    out = (
        jnp.dot(x_ref[...], w_ref[...], preferred_element_type=jnp.float32)
        + b_ref[...]
    )
    gh_ref[...] = out[:, :3 * h]
    xa_ref[...] = out[:, 3 * h:4 * h]
    xb_ref[...] = out[:, 4 * h:]


def _node_linear(x, wcat, bcat, block_rows, interpret=False):
    n, h = x.shape
    ho = wcat.shape[1]
    row = lambda w: pl.BlockSpec((block_rows, w), lambda i: (i, 0))
    return pl.pallas_call(
        _linear_body,
        grid=(n // block_rows,),
        in_specs=[
            row(h),
            pl.BlockSpec((h, ho), lambda i: (0, 0)),
            pl.BlockSpec((1, ho), lambda i: (0, 0)),
        ],
        out_specs=[row(3 * h), row(h), row(h)],
        out_shape=[
            jax.ShapeDtypeStruct((n, 3 * h), jnp.float32),
            jax.ShapeDtypeStruct((n, h), jnp.float32),
            jax.ShapeDtypeStruct((n, h), jnp.float32),
        ],
        interpret=interpret,
    )(x, wcat, bcat)


def _edge_mlp_body(g_ref, ea_ref, wc_ref, b1_ref, w2_ref, b2_ref, o_ref):
    pre = (
        g_ref[...]
        + jnp.dot(ea_ref[...], wc_ref[...], preferred_element_type=jnp.float32)
        + b1_ref[...]
    )
    hdn = jnp.maximum(pre, 0.0)
    o_ref[...] = (
        jnp.dot(hdn, w2_ref[...], preferred_element_type=jnp.float32) + b2_ref[...]
    )


def _edge_mlp(g, ea, wc, b1, w2, b2, block_rows, interpret=False):
    e, h = ea.shape
    row_spec = pl.BlockSpec((block_rows, h), lambda i: (i, 0))
    full = lambda shape: pl.BlockSpec(shape, lambda i: (0, 0))
    return pl.pallas_call(
        _edge_mlp_body,
        grid=(e // block_rows,),
        in_specs=[
            row_spec, row_spec,
            full((h, h)), full((1, h)), full((h, h)), full((1, h)),
        ],
        out_specs=row_spec,
        out_shape=jax.ShapeDtypeStruct((e, h), jnp.float32),
        interpret=interpret,
    )(g, ea, wc, b1, w2, b2)


def _gru_body(a0_ref, a1_ref, gh_ref, x_ref, wih_ref, bih_ref, o_ref):
    h = x_ref.shape[1]
    agg = a0_ref[...] + a1_ref[...]
    gi = (
        jnp.dot(agg, wih_ref[...], preferred_element_type=jnp.float32)
        + bih_ref[...]
    )
    gh = gh_ref[...]
    r = jax.nn.sigmoid(gi[:, :h] + gh[:, :h])
    z = jax.nn.sigmoid(gi[:, h:2 * h] + gh[:, h:2 * h])
    nn = jnp.tanh(gi[:, 2 * h:] + r * gh[:, 2 * h:])
    o_ref[...] = (1.0 - z) * nn + z * x_ref[...]


def _gru(a0, a1, gh, x, wih, bih, block_rows, interpret=False):
    n, h = x.shape
    row_spec = pl.BlockSpec((block_rows, h), lambda i: (i, 0))
    row3_spec = pl.BlockSpec((block_rows, 3 * h), lambda i: (i, 0))
    full = lambda shape: pl.BlockSpec(shape, lambda i: (0, 0))
    return pl.pallas_call(
        _gru_body,
        grid=(n // block_rows,),
        in_specs=[
            row_spec, row_spec, row3_spec, row_spec,
            full((h, 3 * h)), full((1, 3 * h)),
        ],
        out_specs=row_spec,
        out_shape=jax.ShapeDtypeStruct((n, h), jnp.float32),
        interpret=interpret,
    )(a0, a1, gh, x, wih, bih)


def _make_gather(n, e, h):
    epw = e // _NW
    nch = epw // _CHUNK
    mesh = plsc.VectorSubcoreMesh(core_axis_name="c", subcore_axis_name="s")

    @functools.partial(
        pl.kernel,
        out_type=jax.ShapeDtypeStruct((e, h), jnp.float32),
        mesh=mesh,
        scratch_types=[
            pltpu.VMEM((epw,), jnp.int32),
            pltpu.VMEM((epw,), jnp.int32),
            pltpu.VMEM((_CHUNK, h), jnp.float32),
            pltpu.VMEM((_CHUNK, h), jnp.float32),
            pltpu.SemaphoreType.DMA,
            pltpu.SemaphoreType.DMA,
        ],
    )
    def gather(xa_hbm, xb_hbm, src_hbm, dst_hbm, g_hbm,
               sidx, didx, srows, drows, sem_a, sem_b):
        wid = lax.axis_index("s") * _NC + lax.axis_index("c")
        base0 = wid * epw
        pltpu.sync_copy(src_hbm.at[pl.ds(base0, epw)], sidx)
        pltpu.sync_copy(dst_hbm.at[pl.ds(base0, epw)], didx)

        @pl.loop(0, nch)
        def _chunk(j):
            off = j * _CHUNK
            ca = pltpu.async_copy(
                xa_hbm.at[sidx.at[pl.ds(off, _CHUNK)]], srows, sem_a)
            cb = pltpu.async_copy(
                xb_hbm.at[didx.at[pl.ds(off, _CHUNK)]], drows, sem_b)
            ca.wait()
            cb.wait()
            @pl.loop(0, _CHUNK)
            def _row(r):
                for c in range(h // 16):
                    sl = pl.ds(c * 16, 16)
                    srows[r, sl] += drows[r, sl]

            pltpu.sync_copy(srows, g_hbm.at[pl.ds(base0 + off, _CHUNK)])

    return gather


def _make_scatter(n, e, h):
    epw = e // _NW
    nch = epw // _CHUNK
    rpt = (n // _NS) & ~7
    tail = n - _NS * rpt
    mesh = plsc.VectorSubcoreMesh(core_axis_name="c", subcore_axis_name="s")

    @functools.partial(
        pl.kernel,
        out_type=jax.ShapeDtypeStruct((_NC * n, h), jnp.float32),
        mesh=mesh,
        scratch_types=[
            pltpu.VMEM((_CHUNK,), jnp.int32),
            pltpu.VMEM((_CHUNK, h), jnp.float32),
            pltpu.VMEM_SHARED((n, h), jnp.float32),
        ],
    )
    def scatter(msg_hbm, dst_hbm, zero_hbm, out_hbm, idx, rows, acc):
        cid = lax.axis_index("c")
        sid = lax.axis_index("s")
        wid = sid * _NC + cid
        base0 = wid * epw
        pltpu.sync_copy(
            zero_hbm.at[pl.ds(sid * rpt, rpt)],
            acc.at[pl.ds(sid * rpt, rpt)],
        )
        if tail:
            @pl.when(sid == 0)
            def _zero_tail():
                pltpu.sync_copy(
                    zero_hbm.at[pl.ds(_NS * rpt, tail)],
                    acc.at[pl.ds(_NS * rpt, tail)],
                )
        plsc.subcore_barrier()

        @pl.loop(0, nch)
        def _chunk(j):
            base = base0 + j * _CHUNK
            pltpu.sync_copy(dst_hbm.at[pl.ds(base, _CHUNK)], idx)
            pltpu.sync_copy(msg_hbm.at[pl.ds(base, _CHUNK)], rows)
            pltpu.sync_copy(rows, acc.at[idx], add=True)

        plsc.subcore_barrier()
        pltpu.sync_copy(
            acc.at[pl.ds(sid * rpt, rpt)],
            out_hbm.at[pl.ds(cid * n + sid * rpt, rpt)],
        )
        if tail:
            @pl.when(sid == 0)
            def _out_tail():
                pltpu.sync_copy(
                    acc.at[pl.ds(_NS * rpt, tail)],
                    out_hbm.at[pl.ds(cid * n + _NS * rpt, tail)],
                )

    return scatter


def kernel(x, edge_index, edge_attr, W1, b1, W2, b2, Wih, Whh, bih, bhh):
    n, h = x.shape
    e = edge_index.shape[1]
    src = edge_index[0].astype(jnp.int32)
    dst = edge_index[1].astype(jnp.int32)

    wa = W1[:, :h].T
    wb = W1[:, h:2 * h].T
    wc = W1[:, 2 * h:].T
    w2t = W2.T
    whht = Whh.T
    wiht = Wih.T
    wcat = jnp.concatenate([whht, wa, wb], axis=1)
    bcat = jnp.concatenate(
        [bhh, jnp.zeros((2 * h,), jnp.float32)]
    ).reshape(1, 5 * h)

    gh, xa, xb = _node_linear(x, wcat, bcat, block_rows=2000)

    g = _make_gather(n, e, h)(xa, xb, src, dst)

    msg = _edge_mlp(
        g, edge_attr, wc, b1.reshape(1, h), w2t, b2.reshape(1, h),
        block_rows=2000,
    )

    zeros = jnp.zeros((n, h), jnp.float32)
    agg2 = _make_scatter(n, e, h)(msg, dst, zeros)

    return _gru(
        agg2[:n], agg2[n:], gh, x, wiht, bih.reshape(1, 3 * h),
        block_rows=2000,
    )


# speedup vs baseline: 4.8663x; 1.4556x over previous
import functools

import jax
import jax.numpy as jnp
from jax import lax
from jax.experimental import pallas as pl
from jax.experimental.pallas import tpu as pltpu
from jax.experimental.pallas import tpu_sc as plsc

_NC = 2
_NS = 16
_NW = _NC * _NS
_CHUNK = 80


def _linear_body(x_ref, w_ref, b_ref, gh_ref, xa_ref, xb_ref):
    h = x_ref.shape[1]
    out = (
        jnp.dot(x_ref[...], w_ref[...], preferred_element_type=jnp.float32)
        + b_ref[...]
    )
    gh_ref[...] = out[:, :3 * h]
    xa_ref[...] = out[:, 3 * h:4 * h]
    xb_ref[...] = out[:, 4 * h:]


def _node_linear(x, wcat, bcat, block_rows, interpret=False):
    n, h = x.shape
    ho = wcat.shape[1]
    row = lambda w: pl.BlockSpec((block_rows, w), lambda i: (i, 0))
    return pl.pallas_call(
        _linear_body,
        grid=(n // block_rows,),
        in_specs=[
            row(h),
            pl.BlockSpec((h, ho), lambda i: (0, 0)),
            pl.BlockSpec((1, ho), lambda i: (0, 0)),
        ],
        out_specs=[row(3 * h), row(h), row(h)],
        out_shape=[
            jax.ShapeDtypeStruct((n, 3 * h), jnp.float32),
            jax.ShapeDtypeStruct((n, h), jnp.float32),
            jax.ShapeDtypeStruct((n, h), jnp.float32),
        ],
        interpret=interpret,
    )(x, wcat, bcat)


def _edge_mlp_body(g_ref, ea_ref, wc_ref, b1_ref, w2_ref, b2_ref, o_ref):
    pre = (
        g_ref[...]
        + jnp.dot(ea_ref[...], wc_ref[...], preferred_element_type=jnp.float32)
        + b1_ref[...]
    )
    hdn = jnp.maximum(pre, 0.0)
    o_ref[...] = (
        jnp.dot(hdn, w2_ref[...], preferred_element_type=jnp.float32) + b2_ref[...]
    )


def _edge_mlp(g, ea, wc, b1, w2, b2, block_rows, interpret=False):
    e, h = ea.shape
    row_spec = pl.BlockSpec((block_rows, h), lambda i: (i, 0))
    full = lambda shape: pl.BlockSpec(shape, lambda i: (0, 0))
    return pl.pallas_call(
        _edge_mlp_body,
        grid=(e // block_rows,),
        in_specs=[
            row_spec, row_spec,
            full((h, h)), full((1, h)), full((h, h)), full((1, h)),
        ],
        out_specs=row_spec,
        out_shape=jax.ShapeDtypeStruct((e, h), jnp.float32),
        interpret=interpret,
    )(g, ea, wc, b1, w2, b2)


def _gru_body(a0_ref, a1_ref, gh_ref, x_ref, wih_ref, bih_ref, o_ref):
    h = x_ref.shape[1]
    agg = a0_ref[...] + a1_ref[...]
    gi = (
        jnp.dot(agg, wih_ref[...], preferred_element_type=jnp.float32)
        + bih_ref[...]
    )
    gh = gh_ref[...]
    r = jax.nn.sigmoid(gi[:, :h] + gh[:, :h])
    z = jax.nn.sigmoid(gi[:, h:2 * h] + gh[:, h:2 * h])
    nn = jnp.tanh(gi[:, 2 * h:] + r * gh[:, 2 * h:])
    o_ref[...] = (1.0 - z) * nn + z * x_ref[...]


def _gru(a0, a1, gh, x, wih, bih, block_rows, interpret=False):
    n, h = x.shape
    row_spec = pl.BlockSpec((block_rows, h), lambda i: (i, 0))
    row3_spec = pl.BlockSpec((block_rows, 3 * h), lambda i: (i, 0))
    full = lambda shape: pl.BlockSpec(shape, lambda i: (0, 0))
    return pl.pallas_call(
        _gru_body,
        grid=(n // block_rows,),
        in_specs=[
            row_spec, row_spec, row3_spec, row_spec,
            full((h, 3 * h)), full((1, 3 * h)),
        ],
        out_specs=row_spec,
        out_shape=jax.ShapeDtypeStruct((n, h), jnp.float32),
        interpret=interpret,
    )(a0, a1, gh, x, wih, bih)


def _make_gather(n, e, h):
    epw = e // _NW
    nch = epw // _CHUNK
    mesh = plsc.VectorSubcoreMesh(core_axis_name="c", subcore_axis_name="s")

    @functools.partial(
        pl.kernel,
        out_type=jax.ShapeDtypeStruct((e, h), jnp.float32),
        mesh=mesh,
        scratch_types=[
            pltpu.VMEM((epw,), jnp.int32),
            pltpu.VMEM((epw,), jnp.int32),
            pltpu.VMEM((2, _CHUNK, h), jnp.float32),
            pltpu.VMEM((2, _CHUNK, h), jnp.float32),
            pltpu.SemaphoreType.DMA,
            pltpu.SemaphoreType.DMA,
            pltpu.SemaphoreType.DMA,
            pltpu.SemaphoreType.DMA,
        ],
    )
    def gather(xa_hbm, xb_hbm, src_hbm, dst_hbm, g_hbm,
               sidx, didx, srows, drows, gsem0, gsem1, wsem0, wsem1):
        wid = lax.axis_index("s") * _NC + lax.axis_index("c")
        base0 = wid * epw
        pltpu.sync_copy(src_hbm.at[pl.ds(base0, epw)], sidx)
        pltpu.sync_copy(dst_hbm.at[pl.ds(base0, epw)], didx)
        gsems = (gsem0, gsem1)
        wsems = (wsem0, wsem1)

        def start_gather(j, b):
            off = j * _CHUNK
            pltpu.async_copy(
                xa_hbm.at[sidx.at[pl.ds(off, _CHUNK)]], srows.at[b], gsems[b])
            pltpu.async_copy(
                xb_hbm.at[didx.at[pl.ds(off, _CHUNK)]], drows.at[b], gsems[b])

        def wait_gather(b):
            pltpu.make_async_copy(
                xa_hbm.at[sidx.at[pl.ds(0, _CHUNK)]], srows.at[b],
                gsems[b]).wait()
            pltpu.make_async_copy(
                xa_hbm.at[sidx.at[pl.ds(0, _CHUNK)]], drows.at[b],
                gsems[b]).wait()

        def wait_wb(b):
            pltpu.make_async_copy(
                srows.at[b], g_hbm.at[pl.ds(base0, _CHUNK)], wsems[b]).wait()

        def body(j, b, first=False):
            @pl.when(j + 1 < nch)
            def _prefetch():
                if not first:
                    wait_wb(1 - b)
                start_gather(j + 1, 1 - b)
            wait_gather(b)

            @pl.loop(0, _CHUNK)
            def _row(r):
                for c in range(h // 16):
                    sl = pl.ds(c * 16, 16)
                    srows[b, r, sl] += drows[b, r, sl]

            pltpu.async_copy(
                srows.at[b], g_hbm.at[pl.ds(base0 + j * _CHUNK, _CHUNK)],
                wsems[b])

        start_gather(0, 0)
        body(0, 0, first=True)

        @pl.loop(1, nch - 1, step=2)
        def _pair(j):
            body(j, 1)
            body(j + 1, 0)

        wait_wb(0)
        wait_wb(1)

    return gather


def _make_scatter(n, e, h):
    epw = e // _NW
    nch = epw // _CHUNK
    rpt = (n // _NS) & ~7
    tail = n - _NS * rpt
    mesh = plsc.VectorSubcoreMesh(core_axis_name="c", subcore_axis_name="s")

    @functools.partial(
        pl.kernel,
        out_type=jax.ShapeDtypeStruct((_NC * n, h), jnp.float32),
        mesh=mesh,
        scratch_types=[
            pltpu.VMEM((nch, _CHUNK), jnp.int32),
            pltpu.VMEM((2, _CHUNK, h), jnp.float32),
            pltpu.VMEM_SHARED((n, h), jnp.float32),
            pltpu.SemaphoreType.DMA,
            pltpu.SemaphoreType.DMA,
        ],
    )
    def scatter(msg_hbm, dst3d_hbm, zero_hbm, out_hbm, idx, rows, acc,
                lsem0, lsem1):
        cid = lax.axis_index("c")
        sid = lax.axis_index("s")
        wid = sid * _NC + cid
        base0 = wid * epw
        lsems = (lsem0, lsem1)
        pltpu.sync_copy(dst3d_hbm.at[wid], idx)
        pltpu.sync_copy(
            zero_hbm.at[pl.ds(sid * rpt, rpt)],
            acc.at[pl.ds(sid * rpt, rpt)],
        )
        if tail:
            @pl.when(sid == 0)
            def _zero_tail():
                pltpu.sync_copy(
                    zero_hbm.at[pl.ds(_NS * rpt, tail)],
                    acc.at[pl.ds(_NS * rpt, tail)],
                )
        plsc.subcore_barrier()

        def start_load(j, b):
            pltpu.async_copy(
                msg_hbm.at[pl.ds(base0 + j * _CHUNK, _CHUNK)], rows.at[b],
                lsems[b])

        def wait_load(b):
            pltpu.make_async_copy(
                msg_hbm.at[pl.ds(base0, _CHUNK)], rows.at[b], lsems[b]).wait()

        def body(j, b):
            @pl.when(j + 1 < nch)
            def _prefetch():
                start_load(j + 1, 1 - b)
            wait_load(b)
            pltpu.sync_copy(rows.at[b], acc.at[idx.at[j]], add=True)

        start_load(0, 0)
        body(0, 0)

        @pl.loop(1, nch - 1, step=2)
        def _pair(j):
            body(j, 1)
            body(j + 1, 0)

        plsc.subcore_barrier()
        pltpu.sync_copy(
            acc.at[pl.ds(sid * rpt, rpt)],
            out_hbm.at[pl.ds(cid * n + sid * rpt, rpt)],
        )
        if tail:
            @pl.when(sid == 0)
            def _out_tail():
                pltpu.sync_copy(
                    acc.at[pl.ds(_NS * rpt, tail)],
                    out_hbm.at[pl.ds(cid * n + _NS * rpt, tail)],
                )

    return scatter


def kernel(x, edge_index, edge_attr, W1, b1, W2, b2, Wih, Whh, bih, bhh):
    n, h = x.shape
    e = edge_index.shape[1]
    src = edge_index[0].astype(jnp.int32)
    dst = edge_index[1].astype(jnp.int32)

    wa = W1[:, :h].T
    wb = W1[:, h:2 * h].T
    wc = W1[:, 2 * h:].T
    w2t = W2.T
    whht = Whh.T
    wiht = Wih.T
    wcat = jnp.concatenate([whht, wa, wb], axis=1)
    bcat = jnp.concatenate(
        [bhh, jnp.zeros((2 * h,), jnp.float32)]
    ).reshape(1, 5 * h)

    gh, xa, xb = _node_linear(x, wcat, bcat, block_rows=2000)

    g = _make_gather(n, e, h)(xa, xb, src, dst)

    msg = _edge_mlp(
        g, edge_attr, wc, b1.reshape(1, h), w2t, b2.reshape(1, h),
        block_rows=2000,
    )

    zeros = jnp.zeros((n, h), jnp.float32)
    dst3d = dst.reshape(_NW, e // (_NW * _CHUNK), _CHUNK)
    agg2 = _make_scatter(n, e, h)(msg, dst3d, zeros)

    return _gru(
        agg2[:n], agg2[n:], gh, x, wiht, bih.reshape(1, 3 * h),
        block_rows=2000,
    )
